# baseline (speedup 1.0000x reference)
import functools

import jax
import jax.numpy as jnp
from jax import lax
from jax.experimental import pallas as pl
from jax.experimental.pallas import tpu as pltpu
from jax.experimental.pallas import tpu_sc as plsc

N = 10000
E = 320000
D = 128
F = 192
NS = 16
NC = 2

CHUNK = 32
E_PAD = ((E + NS * CHUNK * 6 - 1) // (NS * CHUNK * 6)) * (NS * CHUNK * 6)
EPT = E_PAD // NS
NCHUNK = EPT // CHUNK
N_ACC = 10016
ZPT = N_ACC // NS
OPT = N // NS

CHUNK_D = 128
E_PAD_D = ((E + NS * CHUNK_D - 1) // (NS * CHUNK_D)) * (NS * CHUNK_D)
NCHUNK_D = E_PAD_D // NS // CHUNK_D
N_ACC_D = 10240
ROWS_PT_D = N_ACC_D // NS

BLK = 2000
DELTA = 0.01

_sc_mesh = plsc.VectorSubcoreMesh(core_axis_name="c", subcore_axis_name="s")


def _sc_spmm_body(ec_hbm, y0_hbm, y1_hbm, zero_hbm,
                  out0, out1,
                  ib0, ib1, ib2, ib3, ib4, ib5, gb0, gb1, gb2,
                  y_sp, acc,
                  si0, si1, si2, si3, si4, si5, sg0, sg1, sg2,
                  ss0, ss1, ss2):
    c = lax.axis_index("c")
    s = lax.axis_index("s")
    ibufs = (ib0, ib1, ib2, ib3, ib4, ib5)
    sis = (si0, si1, si2, si3, si4, si5)
    gbufs = (gb0, gb1, gb2)
    sgs = (sg0, sg1, sg2)
    sss = (ss0, ss1, ss2)
    niter = NCHUNK // 6

    pltpu.sync_copy(zero_hbm, acc.at[pl.ds(s * ZPT, ZPT)])

    @pl.when(c == 0)
    def _():
        pltpu.sync_copy(y0_hbm.at[pl.ds(s * OPT, OPT)],
                        y_sp.at[pl.ds(s * OPT, OPT)])

    @pl.when(c == 1)
    def _():
        pltpu.sync_copy(y1_hbm.at[pl.ds(s * OPT, OPT)],
                        y_sp.at[pl.ds(s * OPT, OPT)])

    plsc.subcore_barrier()

    def run(y_hbm):
        def ysrc(par):
            del par
            return y_sp

        for q in range(4):
            pltpu.async_copy(ec_hbm.at[s, q], ibufs[q], sis[q])
        for k0 in range(2):
            pltpu.make_async_copy(
                ec_hbm.at[s, k0], ibufs[k0], sis[k0]).wait()
            pltpu.async_copy(
                ysrc(k0).at[ibufs[k0].at[0]], gbufs[k0], sgs[k0])

        def body(i, carry):
            for j6 in range(6):
                k = 6 * i + j6
                g = j6 % 3
                gp = (j6 + 2) % 3
                qp = (j6 + 5) % 6
                g2 = (j6 + 2) % 3
                q2 = (j6 + 2) % 6
                q4 = (j6 + 4) % 6
                pltpu.make_async_copy(
                    ysrc(j6).at[ibufs[j6].at[0]], gbufs[g], sgs[g]).wait()
                pltpu.async_copy(
                    gbufs[g], acc.at[ibufs[j6].at[1]], sss[g], add=True)

                def drain_prev():
                    pltpu.make_async_copy(
                        gbufs[gp], acc.at[ibufs[qp].at[1]], sss[gp]).wait()

                if j6 == 0:
                    @pl.when(i >= 1)
                    def _():
                        drain_prev()
                else:
                    drain_prev()

                def issue_gather2():
                    pltpu.make_async_copy(
                        ec_hbm.at[s, k + 2], ibufs[q2], sis[q2]).wait()
                    pltpu.async_copy(
                        ysrc(j6 + 2).at[ibufs[q2].at[0]], gbufs[g2], sgs[g2])

                if j6 < 4:
                    issue_gather2()
                else:
                    @pl.when(k + 2 < NCHUNK)
                    def _():
                        issue_gather2()

                if j6 < 2:
                    pltpu.async_copy(ec_hbm.at[s, k + 4], ibufs[q4], sis[q4])
                else:
                    @pl.when(k + 4 < NCHUNK)
                    def _():
                        pltpu.async_copy(
                            ec_hbm.at[s, k + 4], ibufs[q4], sis[q4])
            return carry

        lax.fori_loop(0, niter, body, 0)
        pltpu.make_async_copy(
            gbufs[(NCHUNK - 1) % 3],
            acc.at[ibufs[(NCHUNK - 1) % 6].at[1]],
            sss[(NCHUNK - 1) % 3]).wait()

    @pl.when(c == 0)
    def _():
        run(y0_hbm)

    @pl.when(c == 1)
    def _():
        run(y1_hbm)

    plsc.subcore_barrier()

    @pl.when(c == 0)
    def _():
        pltpu.sync_copy(acc.at[pl.ds(s * OPT, OPT)],
                        out0.at[pl.ds(s * OPT, OPT)])

    @pl.when(c == 1)
    def _():
        pltpu.sync_copy(acc.at[pl.ds(s * OPT, OPT)],
                        out1.at[pl.ds(s * OPT, OPT)])


_sc_spmm = pl.kernel(
    _sc_spmm_body,
    mesh=_sc_mesh,
    compiler_params=pltpu.CompilerParams(use_tc_tiling_on_sc=False),
    out_type=[jax.ShapeDtypeStruct((N, F), jnp.bfloat16)] * 2,
    scratch_types=[pltpu.VMEM((2, CHUNK), jnp.int32)] * 6
    + [pltpu.VMEM((CHUNK, F), jnp.bfloat16)] * 3
    + [
        pltpu.VMEM_SHARED((N, F), jnp.bfloat16),
        pltpu.VMEM_SHARED((N_ACC, F), jnp.bfloat16),
    ]
    + [pltpu.SemaphoreType.DMA] * 12,
)


def _sc_deg_body(row_hbm, ones_hbm, zero_hbm, out_deg,
                 rowbuf, onesbuf, acc):
    c = lax.axis_index("c")
    s = lax.axis_index("s")

    pltpu.sync_copy(zero_hbm, acc.at[pl.ds(s * ROWS_PT_D, ROWS_PT_D)])
    pltpu.sync_copy(row_hbm.at[s], rowbuf)
    pltpu.sync_copy(ones_hbm, onesbuf)
    plsc.subcore_barrier()

    def body(k, carry):
        pltpu.sync_copy(onesbuf, acc.at[rowbuf.at[k]], add=True)
        return carry
    lax.fori_loop(0, NCHUNK_D, body, 0)

    plsc.subcore_barrier()

    @pl.when(c == 0)
    def _():
        pltpu.sync_copy(acc.at[pl.ds(s * ROWS_PT_D, ROWS_PT_D)],
                        out_deg.at[pl.ds(s * ROWS_PT_D, ROWS_PT_D)])


_sc_deg = pl.kernel(
    _sc_deg_body,
    mesh=_sc_mesh,
    compiler_params=pltpu.CompilerParams(use_tc_tiling_on_sc=False),
    out_type=jax.ShapeDtypeStruct((N_ACC_D, 16), jnp.float32),
    scratch_types=[
        pltpu.VMEM((NCHUNK_D, CHUNK_D), jnp.int32),
        pltpu.VMEM((CHUNK_D, 16), jnp.float32),
        pltpu.VMEM_SHARED((N_ACC_D, 16), jnp.float32),
    ],
)


def _tc_init_body(x_ref, y0_ref, y1_ref):
    x = x_ref[...]
    y0_ref[...] = jnp.concatenate(
        [jnp.cos(x), x[:, :64]], axis=1).astype(jnp.bfloat16)
    y1_ref[...] = jnp.concatenate(
        [jnp.sin(x), x[:, 64:]], axis=1).astype(jnp.bfloat16)


_tc_init = pl.pallas_call(
    _tc_init_body,
    grid=(N // BLK,),
    in_specs=[pl.BlockSpec((BLK, D), lambda i: (i, 0))],
    out_specs=[pl.BlockSpec((BLK, F), lambda i: (i, 0))] * 2,
    out_shape=[jax.ShapeDtypeStruct((N, F), jnp.bfloat16)] * 2,
)


def _tc_update_body(x_ref, orig_ref, s0_ref, s1_ref, deg_ref,
                    xo_ref, y0_ref, y1_ref):
    x = x_ref[...]
    deg = deg_ref[:, 0:1]
    ndinv = jnp.where(deg > 0, -1.0 / deg, 0.0)
    s0 = s0_ref[...].astype(jnp.float32)
    s1 = s1_ref[...].astype(jnp.float32)
    off_c = ndinv * s0[:, :D]
    off_s = ndinv * s1[:, :D]
    off_x = ndinv * jnp.concatenate([s0[:, D:], s1[:, D:]], axis=1)
    r = jnp.sqrt(off_c * off_c + off_s * off_s)
    xn = x + DELTA * (orig_ref[...] + r * jnp.sin(-(x + off_x)))
    xo_ref[...] = xn
    y0_ref[...] = jnp.concatenate(
        [jnp.cos(xn), xn[:, :64]], axis=1).astype(jnp.bfloat16)
    y1_ref[...] = jnp.concatenate(
        [jnp.sin(xn), xn[:, 64:]], axis=1).astype(jnp.bfloat16)


_tc_update = pl.pallas_call(
    _tc_update_body,
    grid=(N // BLK,),
    in_specs=[
        pl.BlockSpec((BLK, D), lambda i: (i, 0)),
        pl.BlockSpec((BLK, D), lambda i: (i, 0)),
        pl.BlockSpec((BLK, F), lambda i: (i, 0)),
        pl.BlockSpec((BLK, F), lambda i: (i, 0)),
        pl.BlockSpec((BLK, 16), lambda i: (i, 0)),
    ],
    out_specs=[
        pl.BlockSpec((BLK, D), lambda i: (i, 0)),
        pl.BlockSpec((BLK, F), lambda i: (i, 0)),
        pl.BlockSpec((BLK, F), lambda i: (i, 0)),
    ],
    out_shape=[
        jax.ShapeDtypeStruct((N, D), jnp.float32),
        jax.ShapeDtypeStruct((N, F), jnp.bfloat16),
        jax.ShapeDtypeStruct((N, F), jnp.bfloat16),
    ],
)


def _tc_head_body(x_ref, w_ref, b_ref, o_ref):
    xr = jnp.maximum(x_ref[...], 0.0)
    y = jnp.dot(xr, w_ref[...], preferred_element_type=jnp.float32) + b_ref[...]
    o_ref[...] = jax.nn.sigmoid(y)


_tc_head = pl.pallas_call(
    _tc_head_body,
    grid=(N // BLK,),
    in_specs=[
        pl.BlockSpec((BLK, D), lambda i: (i, 0)),
        pl.BlockSpec((D, 16), lambda i: (0, 0)),
        pl.BlockSpec((1, 16), lambda i: (0, 0)),
    ],
    out_specs=pl.BlockSpec((BLK, 16), lambda i: (i, 0)),
    out_shape=jax.ShapeDtypeStruct((N, 16), jnp.float32),
)


def kernel(edge_index, embed, W2, b2):
    row = edge_index[0].astype(jnp.int32)
    col = edge_index[1].astype(jnp.int32)

    pad = E_PAD - E
    row_p = jnp.concatenate([row, jnp.full((pad,), N, jnp.int32)])
    col_p = jnp.concatenate([col, jnp.zeros((pad,), jnp.int32)])
    ec = jnp.stack([col_p.reshape(NS, NCHUNK, CHUNK),
                    row_p.reshape(NS, NCHUNK, CHUNK)], axis=2)

    pad_d = E_PAD_D - E
    row_d = jnp.concatenate(
        [row, jnp.full((pad_d,), N, jnp.int32)]).reshape(NS, NCHUNK_D, CHUNK_D)

    zero_f = jnp.zeros((ZPT, F), jnp.bfloat16)
    zero_16 = jnp.zeros((ROWS_PT_D, 16), jnp.float32)
    ones_16 = jnp.ones((CHUNK_D, 16), jnp.float32)

    deg16 = _sc_deg(row_d, ones_16, zero_16)

    y0, y1 = _tc_init(embed)

    def body(_, carry):
        x, y0, y1 = carry
        s0, s1 = _sc_spmm(ec, y0, y1, zero_f)
        x, y0, y1 = _tc_update(x, embed, s0, s1, deg16)
        return (x, y0, y1)

    x, y0, y1 = lax.fori_loop(0, 20, body, (embed, y0, y1))

    out = _tc_head(x, W2, b2.reshape(1, 16))
    return (out, x)


# speedup vs baseline: 1.0327x; 1.0327x over previous
import functools

import jax
import jax.numpy as jnp
from jax import lax
from jax.experimental import pallas as pl
from jax.experimental.pallas import tpu as pltpu
from jax.experimental.pallas import tpu_sc as plsc

N = 10000
E = 320000
D = 128
F = 192
NS = 16
NC = 2

CHUNK = 24
E_PAD = ((E + NS * CHUNK * 8 - 1) // (NS * CHUNK * 8)) * (NS * CHUNK * 8)
EPT = E_PAD // NS
NCHUNK = EPT // CHUNK
N_ACC = 10016
ZPT = N_ACC // NS
OPT = N // NS

CHUNK_D = 128
E_PAD_D = ((E + NS * CHUNK_D - 1) // (NS * CHUNK_D)) * (NS * CHUNK_D)
NCHUNK_D = E_PAD_D // NS // CHUNK_D
N_ACC_D = 10240
ROWS_PT_D = N_ACC_D // NS

BLK = 1000
DELTA = 0.01

_sc_mesh = plsc.VectorSubcoreMesh(core_axis_name="c", subcore_axis_name="s")


def _sc_spmm_body(ec_hbm, y0_hbm, y1_hbm, zero_hbm,
                  out0, out1,
                  ib0, ib1, ib2, ib3, ib4, ib5, ib6, ib7,
                  gb0, gb1, gb2, gb3,
                  y_sp, acc,
                  si0, si1, si2, si3, si4, si5, si6, si7,
                  sg0, sg1, sg2, sg3, ss0, ss1, ss2, ss3):
    c = lax.axis_index("c")
    s = lax.axis_index("s")
    ibufs = (ib0, ib1, ib2, ib3, ib4, ib5, ib6, ib7)
    sis = (si0, si1, si2, si3, si4, si5, si6, si7)
    gbufs = (gb0, gb1, gb2, gb3)
    sgs = (sg0, sg1, sg2, sg3)
    sss = (ss0, ss1, ss2, ss3)
    niter = NCHUNK // 8

    pltpu.sync_copy(zero_hbm, acc.at[pl.ds(s * ZPT, ZPT)])

    @pl.when(c == 0)
    def _():
        pltpu.sync_copy(y0_hbm.at[pl.ds(s * OPT, OPT)],
                        y_sp.at[pl.ds(s * OPT, OPT)])

    @pl.when(c == 1)
    def _():
        pltpu.sync_copy(y1_hbm.at[pl.ds(s * OPT, OPT)],
                        y_sp.at[pl.ds(s * OPT, OPT)])

    plsc.subcore_barrier()

    def run():
        for q in range(6):
            pltpu.async_copy(ec_hbm.at[s, q], ibufs[q], sis[q])
        for k0 in range(3):
            pltpu.make_async_copy(
                ec_hbm.at[s, k0], ibufs[k0], sis[k0]).wait()
            pltpu.async_copy(
                y_sp.at[ibufs[k0].at[0]], gbufs[k0], sgs[k0])

        def body(i, carry):
            for j8 in range(8):
                k = 8 * i + j8
                g = j8 % 4
                gp = (j8 + 3) % 4
                qp = (j8 + 7) % 8
                g3 = (j8 + 3) % 4
                q3 = (j8 + 3) % 8
                q6 = (j8 + 6) % 8
                pltpu.make_async_copy(
                    y_sp.at[ibufs[j8].at[0]], gbufs[g], sgs[g]).wait()
                pltpu.async_copy(
                    gbufs[g], acc.at[ibufs[j8].at[1]], sss[g], add=True)

                def drain_prev():
                    pltpu.make_async_copy(
                        gbufs[gp], acc.at[ibufs[qp].at[1]], sss[gp]).wait()

                if j8 == 0:
                    @pl.when(i >= 1)
                    def _():
                        drain_prev()
                else:
                    drain_prev()

                def issue_gather3():
                    pltpu.make_async_copy(
                        ec_hbm.at[s, k + 3], ibufs[q3], sis[q3]).wait()
                    pltpu.async_copy(
                        y_sp.at[ibufs[q3].at[0]], gbufs[g3], sgs[g3])

                if j8 < 5:
                    issue_gather3()
                else:
                    @pl.when(k + 3 < NCHUNK)
                    def _():
                        issue_gather3()

                if j8 < 2:
                    pltpu.async_copy(ec_hbm.at[s, k + 6], ibufs[q6], sis[q6])
                else:
                    @pl.when(k + 6 < NCHUNK)
                    def _():
                        pltpu.async_copy(
                            ec_hbm.at[s, k + 6], ibufs[q6], sis[q6])
            return carry

        lax.fori_loop(0, niter, body, 0)
        pltpu.make_async_copy(
            gbufs[(NCHUNK - 1) % 4],
            acc.at[ibufs[(NCHUNK - 1) % 8].at[1]],
            sss[(NCHUNK - 1) % 4]).wait()

    run()

    plsc.subcore_barrier()

    @pl.when(c == 0)
    def _():
        pltpu.sync_copy(acc.at[pl.ds(s * OPT, OPT)],
                        out0.at[pl.ds(s * OPT, OPT)])

    @pl.when(c == 1)
    def _():
        pltpu.sync_copy(acc.at[pl.ds(s * OPT, OPT)],
                        out1.at[pl.ds(s * OPT, OPT)])


_sc_spmm = pl.kernel(
    _sc_spmm_body,
    mesh=_sc_mesh,
    compiler_params=pltpu.CompilerParams(use_tc_tiling_on_sc=False),
    out_type=[jax.ShapeDtypeStruct((N, F), jnp.bfloat16)] * 2,
    scratch_types=[pltpu.VMEM((2, CHUNK), jnp.int32)] * 8
    + [pltpu.VMEM((CHUNK, F), jnp.bfloat16)] * 4
    + [
        pltpu.VMEM_SHARED((N, F), jnp.bfloat16),
        pltpu.VMEM_SHARED((N_ACC, F), jnp.bfloat16),
    ]
    + [pltpu.SemaphoreType.DMA] * 16,
)


def _sc_deg_body(row_hbm, ones_hbm, zero_hbm, out_deg,
                 rowbuf, onesbuf, acc):
    c = lax.axis_index("c")
    s = lax.axis_index("s")

    pltpu.sync_copy(zero_hbm, acc.at[pl.ds(s * ROWS_PT_D, ROWS_PT_D)])
    pltpu.sync_copy(row_hbm.at[s], rowbuf)
    pltpu.sync_copy(ones_hbm, onesbuf)
    plsc.subcore_barrier()

    def body(k, carry):
        pltpu.sync_copy(onesbuf, acc.at[rowbuf.at[k]], add=True)
        return carry
    lax.fori_loop(0, NCHUNK_D, body, 0)

    plsc.subcore_barrier()

    @pl.when(c == 0)
    def _():
        pltpu.sync_copy(acc.at[pl.ds(s * ROWS_PT_D, ROWS_PT_D)],
                        out_deg.at[pl.ds(s * ROWS_PT_D, ROWS_PT_D)])


_sc_deg = pl.kernel(
    _sc_deg_body,
    mesh=_sc_mesh,
    compiler_params=pltpu.CompilerParams(use_tc_tiling_on_sc=False),
    out_type=jax.ShapeDtypeStruct((N_ACC_D, 16), jnp.float32),
    scratch_types=[
        pltpu.VMEM((NCHUNK_D, CHUNK_D), jnp.int32),
        pltpu.VMEM((CHUNK_D, 16), jnp.float32),
        pltpu.VMEM_SHARED((N_ACC_D, 16), jnp.float32),
    ],
)


def _tc_init_body(x_ref, y0_ref, y1_ref):
    x = x_ref[...]
    y0_ref[...] = jnp.concatenate(
        [jnp.cos(x), x[:, :64]], axis=1).astype(jnp.bfloat16)
    y1_ref[...] = jnp.concatenate(
        [jnp.sin(x), x[:, 64:]], axis=1).astype(jnp.bfloat16)


_tc_init = pl.pallas_call(
    _tc_init_body,
    grid=(N // BLK,),
    in_specs=[pl.BlockSpec((BLK, D), lambda i: (i, 0))],
    out_specs=[pl.BlockSpec((BLK, F), lambda i: (i, 0))] * 2,
    out_shape=[jax.ShapeDtypeStruct((N, F), jnp.bfloat16)] * 2,
)


def _tc_update_body(x_ref, orig_ref, s0_ref, s1_ref, deg_ref,
                    xo_ref, y0_ref, y1_ref):
    x = x_ref[...]
    deg = deg_ref[:, 0:1]
    ndinv = jnp.where(deg > 0, -1.0 / deg, 0.0)
    s0 = s0_ref[...].astype(jnp.float32)
    s1 = s1_ref[...].astype(jnp.float32)
    off_c = ndinv * s0[:, :D]
    off_s = ndinv * s1[:, :D]
    off_x = ndinv * jnp.concatenate([s0[:, D:], s1[:, D:]], axis=1)
    r = jnp.sqrt(off_c * off_c + off_s * off_s)
    xn = x + DELTA * (orig_ref[...] + r * jnp.sin(-(x + off_x)))
    xo_ref[...] = xn
    y0_ref[...] = jnp.concatenate(
        [jnp.cos(xn), xn[:, :64]], axis=1).astype(jnp.bfloat16)
    y1_ref[...] = jnp.concatenate(
        [jnp.sin(xn), xn[:, 64:]], axis=1).astype(jnp.bfloat16)


_tc_update = pl.pallas_call(
    _tc_update_body,
    grid=(N // BLK,),
    in_specs=[
        pl.BlockSpec((BLK, D), lambda i: (i, 0)),
        pl.BlockSpec((BLK, D), lambda i: (i, 0)),
        pl.BlockSpec((BLK, F), lambda i: (i, 0)),
        pl.BlockSpec((BLK, F), lambda i: (i, 0)),
        pl.BlockSpec((BLK, 16), lambda i: (i, 0)),
    ],
    out_specs=[
        pl.BlockSpec((BLK, D), lambda i: (i, 0)),
        pl.BlockSpec((BLK, F), lambda i: (i, 0)),
        pl.BlockSpec((BLK, F), lambda i: (i, 0)),
    ],
    out_shape=[
        jax.ShapeDtypeStruct((N, D), jnp.float32),
        jax.ShapeDtypeStruct((N, F), jnp.bfloat16),
        jax.ShapeDtypeStruct((N, F), jnp.bfloat16),
    ],
)


def _tc_head_body(x_ref, w_ref, b_ref, o_ref):
    xr = jnp.maximum(x_ref[...], 0.0)
    y = jnp.dot(xr, w_ref[...], preferred_element_type=jnp.float32) + b_ref[...]
    o_ref[...] = jax.nn.sigmoid(y)


_tc_head = pl.pallas_call(
    _tc_head_body,
    grid=(N // BLK,),
    in_specs=[
        pl.BlockSpec((BLK, D), lambda i: (i, 0)),
        pl.BlockSpec((D, 16), lambda i: (0, 0)),
        pl.BlockSpec((1, 16), lambda i: (0, 0)),
    ],
    out_specs=pl.BlockSpec((BLK, 16), lambda i: (i, 0)),
    out_shape=jax.ShapeDtypeStruct((N, 16), jnp.float32),
)


def kernel(edge_index, embed, W2, b2):
    row = edge_index[0].astype(jnp.int32)
    col = edge_index[1].astype(jnp.int32)

    pad = E_PAD - E
    row_p = jnp.concatenate([row, jnp.full((pad,), N, jnp.int32)])
    col_p = jnp.concatenate([col, jnp.zeros((pad,), jnp.int32)])
    ec = jnp.stack([col_p.reshape(NS, NCHUNK, CHUNK),
                    row_p.reshape(NS, NCHUNK, CHUNK)], axis=2)

    pad_d = E_PAD_D - E
    row_d = jnp.concatenate(
        [row, jnp.full((pad_d,), N, jnp.int32)]).reshape(NS, NCHUNK_D, CHUNK_D)

    zero_f = jnp.zeros((ZPT, F), jnp.bfloat16)
    zero_16 = jnp.zeros((ROWS_PT_D, 16), jnp.float32)
    ones_16 = jnp.ones((CHUNK_D, 16), jnp.float32)

    deg16 = _sc_deg(row_d, ones_16, zero_16)

    y0, y1 = _tc_init(embed)

    def body(_, carry):
        x, y0, y1 = carry
        s0, s1 = _sc_spmm(ec, y0, y1, zero_f)
        x, y0, y1 = _tc_update(x, embed, s0, s1, deg16)
        return (x, y0, y1)

    x, y0, y1 = lax.fori_loop(0, 20, body, (embed, y0, y1))

    out = _tc_head(x, W2, b2.reshape(1, 16))
    return (out, x)


# speedup vs baseline: 1.0356x; 1.0027x over previous
import functools

import jax
import jax.numpy as jnp
from jax import lax
from jax.experimental import pallas as pl
from jax.experimental.pallas import tpu as pltpu
from jax.experimental.pallas import tpu_sc as plsc

N = 10000
E = 320000
D = 128
F = 192
NS = 16
NC = 2

CHUNK = 24
E_PAD = ((E + NS * CHUNK * 8 - 1) // (NS * CHUNK * 8)) * (NS * CHUNK * 8)
EPT = E_PAD // NS
NCHUNK = EPT // CHUNK
N_ACC = 10016
ZPT = N_ACC // NS
OPT = N // NS

CHUNK_D = 128
E_PAD_D = ((E + NS * CHUNK_D - 1) // (NS * CHUNK_D)) * (NS * CHUNK_D)
NCHUNK_D = E_PAD_D // NS // CHUNK_D
N_ACC_D = 10240
ROWS_PT_D = N_ACC_D // NS

BLK = 1000
DELTA = 0.01

_sc_mesh = plsc.VectorSubcoreMesh(core_axis_name="c", subcore_axis_name="s")


def _sc_spmm_body(ec_hbm, y0_hbm, y1_hbm, zero_hbm,
                  out0, out1,
                  ib0, ib1, ib2, ib3, ib4, ib5, ib6, ib7,
                  gb0, gb1, gb2, gb3,
                  y_sp, acc,
                  si0, si1, si2, si3, si4, si5, si6, si7,
                  sg0, sg1, sg2, sg3, ss0, ss1, ss2, ss3):
    c = lax.axis_index("c")
    s = lax.axis_index("s")
    ibufs = (ib0, ib1, ib2, ib3, ib4, ib5, ib6, ib7)
    sis = (si0, si1, si2, si3, si4, si5, si6, si7)
    gbufs = (gb0, gb1, gb2, gb3)
    sgs = (sg0, sg1, sg2, sg3)
    sss = (ss0, ss1, ss2, ss3)
    niter = NCHUNK // 8

    pltpu.async_copy(zero_hbm, acc.at[pl.ds(s * ZPT, ZPT)], ss0)

    @pl.when(c == 0)
    def _():
        pltpu.async_copy(y0_hbm.at[pl.ds(s * OPT, OPT)],
                         y_sp.at[pl.ds(s * OPT, OPT)], ss1)

    @pl.when(c == 1)
    def _():
        pltpu.async_copy(y1_hbm.at[pl.ds(s * OPT, OPT)],
                         y_sp.at[pl.ds(s * OPT, OPT)], ss1)

    pltpu.make_async_copy(zero_hbm, acc.at[pl.ds(s * ZPT, ZPT)], ss0).wait()
    pltpu.make_async_copy(y0_hbm.at[pl.ds(s * OPT, OPT)],
                          y_sp.at[pl.ds(s * OPT, OPT)], ss1).wait()

    plsc.subcore_barrier()

    def run():
        for q in range(6):
            pltpu.async_copy(ec_hbm.at[s, q], ibufs[q], sis[q])
        for k0 in range(3):
            pltpu.make_async_copy(
                ec_hbm.at[s, k0], ibufs[k0], sis[k0]).wait()
            pltpu.async_copy(
                y_sp.at[ibufs[k0].at[0]], gbufs[k0], sgs[k0])

        def body(i, carry):
            for j8 in range(8):
                k = 8 * i + j8
                g = j8 % 4
                gp = (j8 + 3) % 4
                qp = (j8 + 7) % 8
                g3 = (j8 + 3) % 4
                q3 = (j8 + 3) % 8
                q6 = (j8 + 6) % 8
                pltpu.make_async_copy(
                    y_sp.at[ibufs[j8].at[0]], gbufs[g], sgs[g]).wait()
                pltpu.async_copy(
                    gbufs[g], acc.at[ibufs[j8].at[1]], sss[g], add=True)

                def drain_prev():
                    pltpu.make_async_copy(
                        gbufs[gp], acc.at[ibufs[qp].at[1]], sss[gp]).wait()

                if j8 == 0:
                    @pl.when(i >= 1)
                    def _():
                        drain_prev()
                else:
                    drain_prev()

                def issue_gather3():
                    pltpu.make_async_copy(
                        ec_hbm.at[s, k + 3], ibufs[q3], sis[q3]).wait()
                    pltpu.async_copy(
                        y_sp.at[ibufs[q3].at[0]], gbufs[g3], sgs[g3])

                if j8 < 5:
                    issue_gather3()
                else:
                    @pl.when(k + 3 < NCHUNK)
                    def _():
                        issue_gather3()

                if j8 < 2:
                    pltpu.async_copy(ec_hbm.at[s, k + 6], ibufs[q6], sis[q6])
                else:
                    @pl.when(k + 6 < NCHUNK)
                    def _():
                        pltpu.async_copy(
                            ec_hbm.at[s, k + 6], ibufs[q6], sis[q6])
            return carry

        lax.fori_loop(0, niter, body, 0)
        pltpu.make_async_copy(
            gbufs[(NCHUNK - 1) % 4],
            acc.at[ibufs[(NCHUNK - 1) % 8].at[1]],
            sss[(NCHUNK - 1) % 4]).wait()

    run()

    plsc.subcore_barrier()

    @pl.when(c == 0)
    def _():
        pltpu.sync_copy(acc.at[pl.ds(s * OPT, OPT)],
                        out0.at[pl.ds(s * OPT, OPT)])

    @pl.when(c == 1)
    def _():
        pltpu.sync_copy(acc.at[pl.ds(s * OPT, OPT)],
                        out1.at[pl.ds(s * OPT, OPT)])


_sc_spmm = pl.kernel(
    _sc_spmm_body,
    mesh=_sc_mesh,
    compiler_params=pltpu.CompilerParams(use_tc_tiling_on_sc=False),
    out_type=[jax.ShapeDtypeStruct((N, F), jnp.bfloat16)] * 2,
    scratch_types=[pltpu.VMEM((2, CHUNK), jnp.int32)] * 8
    + [pltpu.VMEM((CHUNK, F), jnp.bfloat16)] * 4
    + [
        pltpu.VMEM_SHARED((N, F), jnp.bfloat16),
        pltpu.VMEM_SHARED((N_ACC, F), jnp.bfloat16),
    ]
    + [pltpu.SemaphoreType.DMA] * 16,
)


def _sc_deg_body(row_hbm, ones_hbm, zero_hbm, out_deg,
                 rowbuf, onesbuf, acc):
    c = lax.axis_index("c")
    s = lax.axis_index("s")

    pltpu.sync_copy(zero_hbm, acc.at[pl.ds(s * ROWS_PT_D, ROWS_PT_D)])
    pltpu.sync_copy(row_hbm.at[s], rowbuf)
    pltpu.sync_copy(ones_hbm, onesbuf)
    plsc.subcore_barrier()

    def body(k, carry):
        pltpu.sync_copy(onesbuf, acc.at[rowbuf.at[k]], add=True)
        return carry
    lax.fori_loop(0, NCHUNK_D, body, 0)

    plsc.subcore_barrier()

    @pl.when(c == 0)
    def _():
        pltpu.sync_copy(acc.at[pl.ds(s * ROWS_PT_D, ROWS_PT_D)],
                        out_deg.at[pl.ds(s * ROWS_PT_D, ROWS_PT_D)])


_sc_deg = pl.kernel(
    _sc_deg_body,
    mesh=_sc_mesh,
    compiler_params=pltpu.CompilerParams(use_tc_tiling_on_sc=False),
    out_type=jax.ShapeDtypeStruct((N_ACC_D, 16), jnp.float32),
    scratch_types=[
        pltpu.VMEM((NCHUNK_D, CHUNK_D), jnp.int32),
        pltpu.VMEM((CHUNK_D, 16), jnp.float32),
        pltpu.VMEM_SHARED((N_ACC_D, 16), jnp.float32),
    ],
)


def _tc_init_body(x_ref, y0_ref, y1_ref):
    x = x_ref[...]
    y0_ref[...] = jnp.concatenate(
        [jnp.cos(x), x[:, :64]], axis=1).astype(jnp.bfloat16)
    y1_ref[...] = jnp.concatenate(
        [jnp.sin(x), x[:, 64:]], axis=1).astype(jnp.bfloat16)


_tc_init = pl.pallas_call(
    _tc_init_body,
    grid=(N // BLK,),
    in_specs=[pl.BlockSpec((BLK, D), lambda i: (i, 0))],
    out_specs=[pl.BlockSpec((BLK, F), lambda i: (i, 0))] * 2,
    out_shape=[jax.ShapeDtypeStruct((N, F), jnp.bfloat16)] * 2,
)


def _tc_update_body(x_ref, orig_ref, s0_ref, s1_ref, deg_ref,
                    xo_ref, y0_ref, y1_ref):
    x = x_ref[...]
    deg = deg_ref[:, 0:1]
    ndinv = jnp.where(deg > 0, -1.0 / deg, 0.0)
    s0 = s0_ref[...].astype(jnp.float32)
    s1 = s1_ref[...].astype(jnp.float32)
    off_c = ndinv * s0[:, :D]
    off_s = ndinv * s1[:, :D]
    off_x = ndinv * jnp.concatenate([s0[:, D:], s1[:, D:]], axis=1)
    r = jnp.sqrt(off_c * off_c + off_s * off_s)
    xn = x + DELTA * (orig_ref[...] + r * jnp.sin(-(x + off_x)))
    xo_ref[...] = xn
    y0_ref[...] = jnp.concatenate(
        [jnp.cos(xn), xn[:, :64]], axis=1).astype(jnp.bfloat16)
    y1_ref[...] = jnp.concatenate(
        [jnp.sin(xn), xn[:, 64:]], axis=1).astype(jnp.bfloat16)


_tc_update = pl.pallas_call(
    _tc_update_body,
    grid=(N // BLK,),
    in_specs=[
        pl.BlockSpec((BLK, D), lambda i: (i, 0)),
        pl.BlockSpec((BLK, D), lambda i: (i, 0)),
        pl.BlockSpec((BLK, F), lambda i: (i, 0)),
        pl.BlockSpec((BLK, F), lambda i: (i, 0)),
        pl.BlockSpec((BLK, 16), lambda i: (i, 0)),
    ],
    out_specs=[
        pl.BlockSpec((BLK, D), lambda i: (i, 0)),
        pl.BlockSpec((BLK, F), lambda i: (i, 0)),
        pl.BlockSpec((BLK, F), lambda i: (i, 0)),
    ],
    out_shape=[
        jax.ShapeDtypeStruct((N, D), jnp.float32),
        jax.ShapeDtypeStruct((N, F), jnp.bfloat16),
        jax.ShapeDtypeStruct((N, F), jnp.bfloat16),
    ],
)


def _tc_head_body(x_ref, w_ref, b_ref, o_ref):
    xr = jnp.maximum(x_ref[...], 0.0)
    y = jnp.dot(xr, w_ref[...], preferred_element_type=jnp.float32) + b_ref[...]
    o_ref[...] = jax.nn.sigmoid(y)


_tc_head = pl.pallas_call(
    _tc_head_body,
    grid=(N // BLK,),
    in_specs=[
        pl.BlockSpec((BLK, D), lambda i: (i, 0)),
        pl.BlockSpec((D, 16), lambda i: (0, 0)),
        pl.BlockSpec((1, 16), lambda i: (0, 0)),
    ],
    out_specs=pl.BlockSpec((BLK, 16), lambda i: (i, 0)),
    out_shape=jax.ShapeDtypeStruct((N, 16), jnp.float32),
)


def kernel(edge_index, embed, W2, b2):
    row = edge_index[0].astype(jnp.int32)
    col = edge_index[1].astype(jnp.int32)

    pad = E_PAD - E
    row_p = jnp.concatenate([row, jnp.full((pad,), N, jnp.int32)])
    col_p = jnp.concatenate([col, jnp.zeros((pad,), jnp.int32)])
    ec = jnp.stack([col_p.reshape(NS, NCHUNK, CHUNK),
                    row_p.reshape(NS, NCHUNK, CHUNK)], axis=2)

    pad_d = E_PAD_D - E
    row_d = jnp.concatenate(
        [row, jnp.full((pad_d,), N, jnp.int32)]).reshape(NS, NCHUNK_D, CHUNK_D)

    zero_f = jnp.zeros((ZPT, F), jnp.bfloat16)
    zero_16 = jnp.zeros((ROWS_PT_D, 16), jnp.float32)
    ones_16 = jnp.ones((CHUNK_D, 16), jnp.float32)

    deg16 = _sc_deg(row_d, ones_16, zero_16)

    y0, y1 = _tc_init(embed)

    def body(_, carry):
        x, y0, y1 = carry
        s0, s1 = _sc_spmm(ec, y0, y1, zero_f)
        x, y0, y1 = _tc_update(x, embed, s0, s1, deg16)
        return (x, y0, y1)

    x, y0, y1 = lax.fori_loop(0, 20, body, (embed, y0, y1))

    out = _tc_head(x, W2, b2.reshape(1, 16))
    return (out, x)
